# baseline (speedup 1.0000x reference)
import functools

import jax
import jax.numpy as jnp
from jax import lax
from jax.experimental import pallas as pl
from jax.experimental.pallas import tpu as pltpu
from jax.experimental.pallas import tpu_sc as plsc

BLK = 512
CH = 128
TPW = 256


def _lower_incl(n):
    ri = lax.broadcasted_iota(jnp.int32, (n, n), 0)
    ci = lax.broadcasted_iota(jnp.int32, (n, n), 1)
    return (ci <= ri).astype(jnp.float32)


def _upper_excl(n):
    ri = lax.broadcasted_iota(jnp.int32, (n, n), 0)
    ci = lax.broadcasted_iota(jnp.int32, (n, n), 1)
    return (ri < ci).astype(jnp.float32)


def _router_meta_body(nt_pad, x_ref, rw_ref, xs_ref, pos_ref, te_ref,
                      ei_s, lp_s, hist_s):
    m = pl.program_id(0)
    M, E = hist_s.shape

    @pl.when(m < M)
    def _router():
        x = x_ref[...]
        logits = jnp.dot(x, rw_ref[...], preferred_element_type=jnp.float32)
        iota_e = lax.broadcasted_iota(jnp.int32, logits.shape, 1)
        mx = jnp.max(logits, axis=1, keepdims=True)
        p = 1.0 / jnp.sum(jnp.exp(logits - mx), axis=1)
        amax = jnp.min(jnp.where(logits == mx, iota_e, E), axis=1)
        onehot = (amax[:, None] == iota_e).astype(jnp.float32)
        tri = (lax.broadcasted_iota(jnp.int32, (BLK, BLK), 0)
               > lax.broadcasted_iota(jnp.int32, (BLK, BLK), 1)).astype(
                   jnp.float32)
        ranks = jnp.dot(tri, onehot, preferred_element_type=jnp.float32)
        local_pos = jnp.sum(ranks * onehot, axis=1)
        xs_ref[...] = (x * p[:, None]).astype(jnp.bfloat16)
        ei_s[m, :] = amax
        lp_s[m, :] = local_pos.astype(jnp.int32)
        hist_s[m, :] = jnp.sum(onehot, axis=0)

    @pl.when(m == M)
    def _meta():
        hist = hist_s[...]
        col_cum = jnp.dot(_lower_incl(M), hist,
                          preferred_element_type=jnp.float32,
                          precision=lax.Precision.HIGHEST)
        col_prefix = col_cum - hist
        counts = jnp.sum(hist, axis=0, keepdims=True)
        pcnt = jnp.floor((counts + (BLK - 1)) * (1.0 / BLK)) * BLK
        pstart = jnp.dot(pcnt, _upper_excl(E),
                         preferred_element_type=jnp.float32,
                         precision=lax.Precision.HIGHEST)
        base = pstart + col_prefix

        ei = ei_s[...]
        lp = lp_s[...]
        acc = jnp.zeros(ei.shape, jnp.float32)
        for e in range(E):
            acc = acc + jnp.where(ei == e, base[:, e:e + 1], 0.0)
        pos_ref[...] = (acc.astype(jnp.int32) + lp).reshape(pos_ref.shape)

        bstart = (lax.broadcasted_iota(jnp.int32, (nt_pad, E), 0)
                  * BLK).astype(jnp.float32)
        te = jnp.sum((pstart <= bstart).astype(jnp.int32), axis=1) - 1
        te_ref[...] = jnp.clip(te, 0, E - 1).reshape(1, nt_pad)


def _ffn_body(te_ref, x_ref, wi_ref, wo_ref, out_ref):
    x = x_ref[...]
    h = jnp.dot(x, wi_ref[0], preferred_element_type=jnp.float32)
    h = jnp.maximum(h, 0.0)
    y = jnp.dot(h, wo_ref[0], preferred_element_type=jnp.float32)
    out_ref[...] = y


def _make_sc_dispatch(T, T_pad, D):
    mesh = plsc.VectorSubcoreMesh(core_axis_name="c", subcore_axis_name="s")

    @functools.partial(
        pl.kernel,
        mesh=mesh,
        out_type=jax.ShapeDtypeStruct((T_pad, D), jnp.float32),
        scratch_types=[
            pltpu.VMEM((TPW // CH, CH), jnp.int32),
            pltpu.VMEM((CH, D), jnp.float32),
            pltpu.SemaphoreType.DMA,
        ],
    )
    def dispatch(x_hbm, pos_hbm, xs_hbm, idx_v, rows_v, sem):
        wid = lax.axis_index("s") * 2 + lax.axis_index("c")
        for ch in range(TPW // CH):
            t0 = wid * TPW + ch * CH
            pltpu.sync_copy(pos_hbm.at[pl.ds(t0, CH)], idx_v.at[ch])
            pltpu.sync_copy(x_hbm.at[pl.ds(t0, CH)], rows_v)
            pltpu.async_copy(rows_v, xs_hbm.at[idx_v.at[ch]], sem).wait()

    return dispatch


def _make_sc_combine(T, T_pad, D):
    mesh = plsc.VectorSubcoreMesh(core_axis_name="c", subcore_axis_name="s")

    @functools.partial(
        pl.kernel,
        mesh=mesh,
        out_type=jax.ShapeDtypeStruct((T, D), jnp.float32),
        scratch_types=[
            pltpu.VMEM((TPW // CH, CH), jnp.int32),
            pltpu.VMEM((CH, D), jnp.float32),
            pltpu.SemaphoreType.DMA,
        ],
    )
    def combine(os_hbm, pos_hbm, out_hbm, idx_v, rows_v, sem):
        wid = lax.axis_index("s") * 2 + lax.axis_index("c")
        for ch in range(TPW // CH):
            t0 = wid * TPW + ch * CH
            pltpu.sync_copy(pos_hbm.at[pl.ds(t0, CH)], idx_v.at[ch])
            pltpu.async_copy(os_hbm.at[idx_v.at[ch]], rows_v, sem).wait()
            pltpu.sync_copy(rows_v, out_hbm.at[pl.ds(t0, CH)])

    return combine


def kernel(hidden_states, router_w, wi, wo):
    B, S, D = hidden_states.shape
    E, _, F = wi.shape
    T = B * S
    M = T // BLK
    NT = M + E - 1
    NT_PAD = ((NT + 7) // 8) * 8
    T_pad = NT_PAD * BLK
    x = hidden_states.reshape(T, D)

    xs_packed, pos3, te = pl.pallas_call(
        functools.partial(_router_meta_body, NT_PAD),
        grid=(M + 1,),
        in_specs=[
            pl.BlockSpec((BLK, D), lambda m: (jnp.minimum(m, M - 1), 0)),
            pl.BlockSpec((D, E), lambda m: (0, 0)),
        ],
        out_specs=[
            pl.BlockSpec((BLK, D), lambda m: (jnp.minimum(m, M - 1), 0)),
            pl.BlockSpec((M, 1, BLK), lambda m: (0, 0, 0)),
            pl.BlockSpec((1, NT_PAD), lambda m: (0, 0)),
        ],
        out_shape=[
            jax.ShapeDtypeStruct((T, D), jnp.bfloat16),
            jax.ShapeDtypeStruct((M, 1, BLK), jnp.int32),
            jax.ShapeDtypeStruct((1, NT_PAD), jnp.int32),
        ],
        scratch_shapes=[
            pltpu.VMEM((M, BLK), jnp.int32),
            pltpu.VMEM((M, BLK), jnp.int32),
            pltpu.VMEM((M, E), jnp.float32),
        ],
    )(x, router_w)

    pos = pos3.reshape(T)

    xs_p32 = lax.bitcast_convert_type(
        xs_packed.reshape(T, D // 2, 2), jnp.float32)
    xsorted_p32 = _make_sc_dispatch(T, T_pad, D // 2)(xs_p32, pos)
    x_sorted = lax.bitcast_convert_type(
        xsorted_p32, jnp.bfloat16).reshape(T_pad, D)

    out_sorted = pl.pallas_call(
        _ffn_body,
        grid_spec=pltpu.PrefetchScalarGridSpec(
            num_scalar_prefetch=1,
            grid=(NT_PAD,),
            in_specs=[
                pl.BlockSpec((BLK, D), lambda i, te: (i, 0)),
                pl.BlockSpec((1, D, F), lambda i, te: (te[0, i], 0, 0)),
                pl.BlockSpec((1, F, D), lambda i, te: (te[0, i], 0, 0)),
            ],
            out_specs=pl.BlockSpec((BLK, D), lambda i, te: (i, 0)),
        ),
        out_shape=jax.ShapeDtypeStruct((T_pad, D), jnp.float32),
    )(te, x_sorted, wi.astype(jnp.bfloat16), wo)

    out = _make_sc_combine(T, T_pad, D)(out_sorted, pos)
    return out.reshape(B, S, D)


# speedup vs baseline: 3.0751x; 3.0751x over previous
import functools

import jax
import jax.numpy as jnp
from jax import lax
from jax.experimental import pallas as pl
from jax.experimental.pallas import tpu as pltpu
from jax.experimental.pallas import tpu_sc as plsc

BLK = 512
CH = 128
TPW = 256


def _lower_incl(n):
    ri = lax.broadcasted_iota(jnp.int32, (n, n), 0)
    ci = lax.broadcasted_iota(jnp.int32, (n, n), 1)
    return (ci <= ri).astype(jnp.float32)


def _upper_incl(n):
    ri = lax.broadcasted_iota(jnp.int32, (n, n), 0)
    ci = lax.broadcasted_iota(jnp.int32, (n, n), 1)
    return (ri <= ci).astype(jnp.float32)


def _router_meta_body(nt_pad, x_ref, rw_ref, xs_ref, pos_ref, tm_ref, te_ref,
                      tf_ref, rl_ref, rh_ref, ei_s, lp_s, hist_s):
    m = pl.program_id(0)
    M, E = hist_s.shape

    @pl.when(m < M)
    def _router():
        x = x_ref[...]
        logits = jnp.dot(x, rw_ref[...], preferred_element_type=jnp.float32)
        iota_e = lax.broadcasted_iota(jnp.int32, logits.shape, 1)
        mx = jnp.max(logits, axis=1, keepdims=True)
        p = 1.0 / jnp.sum(jnp.exp(logits - mx), axis=1)
        amax = jnp.min(jnp.where(logits == mx, iota_e, E), axis=1)
        onehot = (amax[:, None] == iota_e).astype(jnp.float32)
        tri = (lax.broadcasted_iota(jnp.int32, (BLK, BLK), 0)
               > lax.broadcasted_iota(jnp.int32, (BLK, BLK), 1)).astype(
                   jnp.float32)
        ranks = jnp.dot(tri, onehot, preferred_element_type=jnp.float32)
        local_pos = jnp.sum(ranks * onehot, axis=1)
        xs_ref[...] = x * p[:, None]
        ei_s[m, :] = amax
        lp_s[m, :] = local_pos.astype(jnp.int32)
        hist_s[m, :] = jnp.sum(onehot, axis=0)

    @pl.when(m == M)
    def _meta():
        hist = hist_s[...]
        col_cum = jnp.dot(_lower_incl(M), hist,
                          preferred_element_type=jnp.float32,
                          precision=lax.Precision.HIGHEST)
        col_prefix = col_cum - hist
        counts = jnp.sum(hist, axis=0, keepdims=True)
        c_end = jnp.dot(counts, _upper_incl(E),
                        preferred_element_type=jnp.float32,
                        precision=lax.Precision.HIGHEST)
        c_excl = c_end - counts
        base = c_excl + col_prefix

        ei = ei_s[...]
        lp = lp_s[...]
        acc = jnp.zeros(ei.shape, jnp.float32)
        for e in range(E):
            acc = acc + jnp.where(ei == e, base[:, e:e + 1], 0.0)
        pos_ref[...] = (acc.astype(jnp.int32) + lp).reshape(pos_ref.shape)

        e_ge1 = lax.broadcasted_iota(jnp.int32, (M, E), 1) >= 1
        m_start = (lax.broadcasted_iota(jnp.int32, (M, E), 0) * BLK).astype(
            jnp.float32)
        ef = jnp.sum(((c_excl <= m_start) & e_ge1).astype(jnp.int32), axis=1)
        el = jnp.sum(((c_excl <= m_start + (BLK - 1)) & e_ge1).astype(
            jnp.int32), axis=1)
        cnt = (el - ef + 1).reshape(1, M).astype(jnp.float32)
        st_incl = jnp.dot(cnt, _upper_incl(M),
                          preferred_element_type=jnp.float32,
                          precision=lax.Precision.HIGHEST)
        st = (st_incl - cnt).astype(jnp.int32)
        nt_act = jnp.sum(cnt.astype(jnp.int32))

        ti = lax.broadcasted_iota(jnp.int32, (nt_pad, M), 0)
        m_i = jnp.sum((st <= ti).astype(jnp.int32), axis=1) - 1
        onehot_m = (m_i[:, None] == lax.broadcasted_iota(
            jnp.int32, (nt_pad, M), 1)).astype(jnp.int32)
        ef_g = jnp.sum(onehot_m * ef[None, :], axis=1)
        st_g = jnp.sum(onehot_m * st, axis=1)
        i_vec = jnp.max(ti, axis=1)
        e_i = jnp.clip(ef_g + (i_vec - st_g), 0, E - 1)
        active = i_vec < nt_act
        first = ((i_vec == st_g) & active).astype(jnp.int32)
        onehot_e = (e_i[:, None] == lax.broadcasted_iota(
            jnp.int32, (nt_pad, E), 1)).astype(jnp.float32)
        ce_g = jnp.sum(onehot_e * c_excl, axis=1)
        cend_g = jnp.sum(onehot_e * c_end, axis=1)
        m_base = (m_i * BLK).astype(jnp.float32)
        lo = jnp.maximum(ce_g, m_base) - m_base
        hi = jnp.minimum(cend_g, m_base + BLK) - m_base
        lo = jnp.where(active, lo, 0.0).astype(jnp.int32)
        hi = jnp.where(active, hi, 0.0).astype(jnp.int32)
        tm_ref[...] = m_i.reshape(1, nt_pad)
        te_ref[...] = e_i.reshape(1, nt_pad)
        tf_ref[...] = first.reshape(1, nt_pad)
        rl_ref[...] = lo.reshape(1, nt_pad)
        rh_ref[...] = hi.reshape(1, nt_pad)


def _ffn_body(tm_ref, te_ref, tf_ref, rl_ref, rh_ref,
              x_ref, wi_ref, wo_ref, out_ref):
    i = pl.program_id(0)
    lo = rl_ref[0, i]
    hi = rh_ref[0, i]
    first = tf_ref[0, i]
    r = lax.broadcasted_iota(jnp.int32, (BLK, 1), 0)
    mask = (r >= lo) & (r < hi)
    x = x_ref[...]
    h = jnp.dot(x, wi_ref[0], preferred_element_type=jnp.float32)
    h = jnp.where(mask, jnp.maximum(h, 0.0), 0.0)
    y = jnp.dot(h, wo_ref[0], preferred_element_type=jnp.float32)

    @pl.when(first == 1)
    def _():
        out_ref[...] = y

    @pl.when(first == 0)
    def _():
        out_ref[...] += y


def _make_sc_dispatch(T, D):
    mesh = plsc.VectorSubcoreMesh(core_axis_name="c", subcore_axis_name="s")

    @functools.partial(
        pl.kernel,
        mesh=mesh,
        out_type=jax.ShapeDtypeStruct((T, D), jnp.float32),
        scratch_types=[
            pltpu.VMEM((TPW // CH, CH), jnp.int32),
            pltpu.VMEM((CH, D), jnp.float32),
            pltpu.SemaphoreType.DMA,
        ],
    )
    def dispatch(x_hbm, pos_hbm, xs_hbm, idx_v, rows_v, sem):
        wid = lax.axis_index("s") * 2 + lax.axis_index("c")
        for ch in range(TPW // CH):
            t0 = wid * TPW + ch * CH
            pltpu.sync_copy(pos_hbm.at[pl.ds(t0, CH)], idx_v.at[ch])
            pltpu.sync_copy(x_hbm.at[pl.ds(t0, CH)], rows_v)
            pltpu.async_copy(rows_v, xs_hbm.at[idx_v.at[ch]], sem).wait()

    return dispatch


def _make_sc_combine(T, D):
    mesh = plsc.VectorSubcoreMesh(core_axis_name="c", subcore_axis_name="s")

    @functools.partial(
        pl.kernel,
        mesh=mesh,
        out_type=jax.ShapeDtypeStruct((T, D), jnp.float32),
        scratch_types=[
            pltpu.VMEM((TPW // CH, CH), jnp.int32),
            pltpu.VMEM((CH, D), jnp.float32),
            pltpu.SemaphoreType.DMA,
        ],
    )
    def combine(os_hbm, pos_hbm, out_hbm, idx_v, rows_v, sem):
        wid = lax.axis_index("s") * 2 + lax.axis_index("c")
        for ch in range(TPW // CH):
            t0 = wid * TPW + ch * CH
            pltpu.sync_copy(pos_hbm.at[pl.ds(t0, CH)], idx_v.at[ch])
            pltpu.async_copy(os_hbm.at[idx_v.at[ch]], rows_v, sem).wait()
            pltpu.sync_copy(rows_v, out_hbm.at[pl.ds(t0, CH)])

    return combine


def kernel(hidden_states, router_w, wi, wo):
    B, S, D = hidden_states.shape
    E, _, F = wi.shape
    T = B * S
    M = T // BLK
    NT = M + E - 1
    NT_PAD = ((NT + 7) // 8) * 8
    x = hidden_states.reshape(T, D)

    xs, pos3, tm, te, tf, rl, rh = pl.pallas_call(
        functools.partial(_router_meta_body, NT_PAD),
        grid=(M + 1,),
        in_specs=[
            pl.BlockSpec((BLK, D), lambda m: (jnp.minimum(m, M - 1), 0)),
            pl.BlockSpec((D, E), lambda m: (0, 0)),
        ],
        out_specs=[
            pl.BlockSpec((BLK, D), lambda m: (jnp.minimum(m, M - 1), 0)),
            pl.BlockSpec((M, 1, BLK), lambda m: (0, 0, 0)),
            pl.BlockSpec((1, NT_PAD), lambda m: (0, 0)),
            pl.BlockSpec((1, NT_PAD), lambda m: (0, 0)),
            pl.BlockSpec((1, NT_PAD), lambda m: (0, 0)),
            pl.BlockSpec((1, NT_PAD), lambda m: (0, 0)),
            pl.BlockSpec((1, NT_PAD), lambda m: (0, 0)),
        ],
        out_shape=[
            jax.ShapeDtypeStruct((T, D), jnp.float32),
            jax.ShapeDtypeStruct((M, 1, BLK), jnp.int32),
            jax.ShapeDtypeStruct((1, NT_PAD), jnp.int32),
            jax.ShapeDtypeStruct((1, NT_PAD), jnp.int32),
            jax.ShapeDtypeStruct((1, NT_PAD), jnp.int32),
            jax.ShapeDtypeStruct((1, NT_PAD), jnp.int32),
            jax.ShapeDtypeStruct((1, NT_PAD), jnp.int32),
        ],
        scratch_shapes=[
            pltpu.VMEM((M, BLK), jnp.int32),
            pltpu.VMEM((M, BLK), jnp.int32),
            pltpu.VMEM((M, E), jnp.float32),
        ],
    )(x, router_w)

    pos = pos3.reshape(T)

    x_sorted = _make_sc_dispatch(T, D)(xs, pos)

    out_sorted = pl.pallas_call(
        _ffn_body,
        grid_spec=pltpu.PrefetchScalarGridSpec(
            num_scalar_prefetch=5,
            grid=(NT_PAD,),
            in_specs=[
                pl.BlockSpec((BLK, D),
                             lambda i, tm, te, tf, rl, rh: (tm[0, i], 0)),
                pl.BlockSpec((1, D, F),
                             lambda i, tm, te, tf, rl, rh: (te[0, i], 0, 0)),
                pl.BlockSpec((1, F, D),
                             lambda i, tm, te, tf, rl, rh: (te[0, i], 0, 0)),
            ],
            out_specs=pl.BlockSpec(
                (BLK, D), lambda i, tm, te, tf, rl, rh: (tm[0, i], 0)),
        ),
        out_shape=jax.ShapeDtypeStruct((T, D), jnp.float32),
    )(tm, te, tf, rl, rh, x_sorted, wi, wo)

    out = _make_sc_combine(T, D)(out_sorted, pos)
    return out.reshape(B, S, D)


# speedup vs baseline: 3.1129x; 1.0123x over previous
import functools

import jax
import jax.numpy as jnp
from jax import lax
from jax.experimental import pallas as pl
from jax.experimental.pallas import tpu as pltpu
from jax.experimental.pallas import tpu_sc as plsc

BLK = 512
CH = 128
TPW = 256


def _lower_incl(n):
    ri = lax.broadcasted_iota(jnp.int32, (n, n), 0)
    ci = lax.broadcasted_iota(jnp.int32, (n, n), 1)
    return (ci <= ri).astype(jnp.float32)


def _upper_incl(n):
    ri = lax.broadcasted_iota(jnp.int32, (n, n), 0)
    ci = lax.broadcasted_iota(jnp.int32, (n, n), 1)
    return (ri <= ci).astype(jnp.float32)


def _router_meta_body(nt_pad, x_ref, rw_ref, xs_ref, pos_ref, tm_ref, te_ref,
                      tf_ref, rl_ref, rh_ref, ei_s, lp_s, hist_s):
    m = pl.program_id(0)
    M, E = hist_s.shape

    @pl.when(m < M)
    def _router():
        x = x_ref[...]
        logits = jnp.dot(x, rw_ref[...], preferred_element_type=jnp.float32)
        iota_e = lax.broadcasted_iota(jnp.int32, logits.shape, 1)
        mx = jnp.max(logits, axis=1, keepdims=True)
        p = 1.0 / jnp.sum(jnp.exp(logits - mx), axis=1)
        amax = jnp.min(jnp.where(logits == mx, iota_e, E), axis=1)
        onehot = (amax[:, None] == iota_e).astype(jnp.float32)
        tri = (lax.broadcasted_iota(jnp.int32, (BLK, BLK), 0)
               > lax.broadcasted_iota(jnp.int32, (BLK, BLK), 1)).astype(
                   jnp.float32)
        ranks = jnp.dot(tri, onehot, preferred_element_type=jnp.float32)
        local_pos = jnp.sum(ranks * onehot, axis=1)
        xs_ref[...] = x * p[:, None]
        ei_s[m, :] = amax
        lp_s[m, :] = local_pos.astype(jnp.int32)
        hist_s[m, :] = jnp.sum(onehot, axis=0)

    @pl.when(m == M)
    def _meta():
        hist = hist_s[...]
        col_cum = jnp.dot(_lower_incl(M), hist,
                          preferred_element_type=jnp.float32,
                          precision=lax.Precision.HIGHEST)
        col_prefix = col_cum - hist
        counts = jnp.sum(hist, axis=0, keepdims=True)
        c_end = jnp.dot(counts, _upper_incl(E),
                        preferred_element_type=jnp.float32,
                        precision=lax.Precision.HIGHEST)
        c_excl = c_end - counts
        base = c_excl + col_prefix

        ei = ei_s[...]
        lp = lp_s[...]
        acc = jnp.zeros(ei.shape, jnp.float32)
        for e in range(E):
            acc = acc + jnp.where(ei == e, base[:, e:e + 1], 0.0)
        pos_ref[...] = (acc.astype(jnp.int32) + lp).reshape(pos_ref.shape)

        e_ge1 = lax.broadcasted_iota(jnp.int32, (M, E), 1) >= 1
        m_start = (lax.broadcasted_iota(jnp.int32, (M, E), 0) * BLK).astype(
            jnp.float32)
        ef = jnp.sum(((c_excl <= m_start) & e_ge1).astype(jnp.int32), axis=1)
        el = jnp.sum(((c_excl <= m_start + (BLK - 1)) & e_ge1).astype(
            jnp.int32), axis=1)
        cnt = (el - ef + 1).reshape(1, M).astype(jnp.float32)
        st_incl = jnp.dot(cnt, _upper_incl(M),
                          preferred_element_type=jnp.float32,
                          precision=lax.Precision.HIGHEST)
        st = (st_incl - cnt).astype(jnp.int32)
        nt_act = jnp.sum(cnt.astype(jnp.int32))

        ti = lax.broadcasted_iota(jnp.int32, (nt_pad, M), 0)
        m_i = jnp.sum((st <= ti).astype(jnp.int32), axis=1) - 1
        onehot_m = (m_i[:, None] == lax.broadcasted_iota(
            jnp.int32, (nt_pad, M), 1)).astype(jnp.int32)
        ef_g = jnp.sum(onehot_m * ef[None, :], axis=1)
        st_g = jnp.sum(onehot_m * st, axis=1)
        i_vec = jnp.max(ti, axis=1)
        e_i = jnp.clip(ef_g + (i_vec - st_g), 0, E - 1)
        active = i_vec < nt_act
        first = ((i_vec == st_g) & active).astype(jnp.int32)
        onehot_e = (e_i[:, None] == lax.broadcasted_iota(
            jnp.int32, (nt_pad, E), 1)).astype(jnp.float32)
        ce_g = jnp.sum(onehot_e * c_excl, axis=1)
        cend_g = jnp.sum(onehot_e * c_end, axis=1)
        m_base = (m_i * BLK).astype(jnp.float32)
        lo = jnp.maximum(ce_g, m_base) - m_base
        hi = jnp.minimum(cend_g, m_base + BLK) - m_base
        lo = jnp.where(active, lo, 0.0).astype(jnp.int32)
        hi = jnp.where(active, hi, 0.0).astype(jnp.int32)
        tm_ref[...] = m_i.reshape(1, nt_pad)
        te_ref[...] = e_i.reshape(1, nt_pad)
        tf_ref[...] = first.reshape(1, nt_pad)
        rl_ref[...] = lo.reshape(1, nt_pad)
        rh_ref[...] = hi.reshape(1, nt_pad)


def _ffn_body(tm_ref, te_ref, tf_ref, rl_ref, rh_ref,
              x_ref, wi_ref, wo_ref, out_ref):
    i = pl.program_id(0)
    lo = rl_ref[0, i]
    hi = rh_ref[0, i]
    first = tf_ref[0, i]
    r = lax.broadcasted_iota(jnp.int32, (BLK, 1), 0)
    mask = (r >= lo) & (r < hi)
    x = x_ref[...]
    h = jnp.dot(x, wi_ref[0], preferred_element_type=jnp.float32)
    h = jnp.where(mask, jnp.maximum(h, 0.0), 0.0)
    y = jnp.dot(h, wo_ref[0], preferred_element_type=jnp.float32)

    @pl.when(first == 1)
    def _():
        out_ref[...] = y

    @pl.when(first == 0)
    def _():
        out_ref[...] += y


def _make_sc_dispatch(T, D):
    mesh = plsc.VectorSubcoreMesh(core_axis_name="c", subcore_axis_name="s")

    @functools.partial(
        pl.kernel,
        mesh=mesh,
        out_type=jax.ShapeDtypeStruct((T, D), jnp.float32),
        scratch_types=[
            pltpu.VMEM((TPW // CH, CH), jnp.int32),
            pltpu.VMEM((CH, D), jnp.float32),
            pltpu.SemaphoreType.DMA,
        ],
    )
    def dispatch(x_hbm, pos_hbm, xs_hbm, idx_v, rows_v, sem):
        wid = lax.axis_index("s") * 2 + lax.axis_index("c")
        for ch in range(TPW // CH):
            t0 = wid * TPW + ch * CH
            pltpu.sync_copy(pos_hbm.at[pl.ds(t0, CH)], idx_v.at[ch])
            pltpu.sync_copy(x_hbm.at[pl.ds(t0, CH)], rows_v)
            pltpu.async_copy(rows_v, xs_hbm.at[idx_v.at[ch]], sem).wait()

    return dispatch


def _make_sc_combine(T, D):
    mesh = plsc.VectorSubcoreMesh(core_axis_name="c", subcore_axis_name="s")

    @functools.partial(
        pl.kernel,
        mesh=mesh,
        out_type=jax.ShapeDtypeStruct((T, D), jnp.float32),
        scratch_types=[
            pltpu.VMEM((TPW // CH, CH), jnp.int32),
            pltpu.VMEM((CH, D), jnp.float32),
            pltpu.SemaphoreType.DMA,
        ],
    )
    def combine(os_hbm, pos_hbm, out_hbm, idx_v, rows_v, sem):
        wid = lax.axis_index("s") * 2 + lax.axis_index("c")
        for ch in range(TPW // CH):
            t0 = wid * TPW + ch * CH
            pltpu.sync_copy(pos_hbm.at[pl.ds(t0, CH)], idx_v.at[ch])
            pltpu.async_copy(os_hbm.at[idx_v.at[ch]], rows_v, sem).wait()
            pltpu.sync_copy(rows_v, out_hbm.at[pl.ds(t0, CH)])

    return combine


def kernel(hidden_states, router_w, wi, wo):
    B, S, D = hidden_states.shape
    E, _, F = wi.shape
    T = B * S
    M = T // BLK
    NT = M + E - 1
    NT_PAD = NT
    x = hidden_states.reshape(T, D)

    xs, pos3, tm, te, tf, rl, rh = pl.pallas_call(
        functools.partial(_router_meta_body, NT_PAD),
        grid=(M + 1,),
        in_specs=[
            pl.BlockSpec((BLK, D), lambda m: (jnp.minimum(m, M - 1), 0)),
            pl.BlockSpec((D, E), lambda m: (0, 0)),
        ],
        out_specs=[
            pl.BlockSpec((BLK, D), lambda m: (jnp.minimum(m, M - 1), 0)),
            pl.BlockSpec((M, 1, BLK), lambda m: (0, 0, 0)),
            pl.BlockSpec((1, NT_PAD), lambda m: (0, 0)),
            pl.BlockSpec((1, NT_PAD), lambda m: (0, 0)),
            pl.BlockSpec((1, NT_PAD), lambda m: (0, 0)),
            pl.BlockSpec((1, NT_PAD), lambda m: (0, 0)),
            pl.BlockSpec((1, NT_PAD), lambda m: (0, 0)),
        ],
        out_shape=[
            jax.ShapeDtypeStruct((T, D), jnp.float32),
            jax.ShapeDtypeStruct((M, 1, BLK), jnp.int32),
            jax.ShapeDtypeStruct((1, NT_PAD), jnp.int32),
            jax.ShapeDtypeStruct((1, NT_PAD), jnp.int32),
            jax.ShapeDtypeStruct((1, NT_PAD), jnp.int32),
            jax.ShapeDtypeStruct((1, NT_PAD), jnp.int32),
            jax.ShapeDtypeStruct((1, NT_PAD), jnp.int32),
        ],
        scratch_shapes=[
            pltpu.VMEM((M, BLK), jnp.int32),
            pltpu.VMEM((M, BLK), jnp.int32),
            pltpu.VMEM((M, E), jnp.float32),
        ],
    )(x, router_w)

    pos = pos3.reshape(T)

    x_sorted = _make_sc_dispatch(T, D)(xs, pos)

    out_sorted = pl.pallas_call(
        _ffn_body,
        grid_spec=pltpu.PrefetchScalarGridSpec(
            num_scalar_prefetch=5,
            grid=(NT_PAD,),
            in_specs=[
                pl.BlockSpec((BLK, D),
                             lambda i, tm, te, tf, rl, rh: (tm[0, i], 0)),
                pl.BlockSpec((1, D, F),
                             lambda i, tm, te, tf, rl, rh: (te[0, i], 0, 0)),
                pl.BlockSpec((1, F, D),
                             lambda i, tm, te, tf, rl, rh: (te[0, i], 0, 0)),
            ],
            out_specs=pl.BlockSpec(
                (BLK, D), lambda i, tm, te, tf, rl, rh: (tm[0, i], 0)),
        ),
        out_shape=jax.ShapeDtypeStruct((T, D), jnp.float32),
    )(tm, te, tf, rl, rh, x_sorted, wi, wo)

    out = _make_sc_combine(T, D)(out_sorted, pos)
    return out.reshape(B, S, D)
